# baseline (speedup 1.0000x reference)
import functools

import jax
import jax.numpy as jnp
from jax import lax
from jax.experimental import pallas as pl
from jax.experimental.pallas import tpu as pltpu
from jax.experimental.pallas import tpu_sc as plsc

VOCAB = 1000000
EMBED = 64
B = 4096
L = 200
EPS = 1e-5

BL = B * L
NC, NS = 2, 16
NW = NC * NS
ROWS_PER_W = BL // NW
CHUNK = 640
NCHUNK = ROWS_PER_W // CHUNK
GSIZE = 128
NG = CHUNK // GSIZE
NBUF = 2
NGRP = CHUNK // 16
PCYC = 2 * L
PBUF = PCYC + 16


def _rsqrt16(v):
    i = lax.bitcast_convert_type(v, jnp.int32)
    y = lax.bitcast_convert_type(
        jnp.int32(0x5F3759DF) - lax.shift_right_arithmetic(i, 1), jnp.float32)
    for _ in range(3):
        y = y * (1.5 - 0.5 * v * y * y)
    return y


def _sc_body(ids_hbm, tok_hbm, posc_hbm, gb_hbm, out_hbm,
             idx_v, rows_v, pos_v, gb_vm, sem0, sem1):
    wid = lax.axis_index("s") * NC + lax.axis_index("c")
    sems = (sem0, sem1)

    pltpu.sync_copy(posc_hbm, pos_v)
    pltpu.sync_copy(gb_hbm, gb_vm)
    gvec = [gb_vm[0, pl.ds(16 * k, 16)] for k in range(4)]
    bvec = [gb_vm[1, pl.ds(16 * k, 16)] for k in range(4)]

    def start_gather(c, b):
        idx_row = wid * (ROWS_PER_W // GSIZE) + c * NG
        pltpu.sync_copy(ids_hbm.at[pl.ds(idx_row, NG)],
                        idx_v.at[pl.ds(b * NG, NG)])
        for j in range(NG):
            pltpu.async_copy(
                tok_hbm.at[idx_v.at[b * NG + j]],
                rows_v.at[pl.ds(b * CHUNK + j * GSIZE, GSIZE)], sems[b])

    def drain(b):
        pltpu.make_async_copy(
            tok_hbm.at[pl.ds(0, CHUNK)],
            rows_v.at[pl.ds(b * CHUNK, CHUNK)], sems[b]).wait()

    iota16 = lax.iota(jnp.int32, 16)

    def compute(c, b):
        def grp_body(gi, carry):
            r0 = b * CHUNK + gi * 16
            ridx = r0 + iota16
            om = lax.rem(c * CHUNK + gi * 16, PCYC)
            s = [jnp.zeros((16,), jnp.float32) for _ in range(4)]
            q = [jnp.zeros((16,), jnp.float32) for _ in range(4)]
            for j in range(EMBED):
                cj = jnp.full((16,), j, jnp.int32)
                x = plsc.load_gather(rows_v, [ridx, cj])
                sv = x + pos_v[j, pl.ds(om, 16)]
                s[j % 4] = s[j % 4] + sv
                q[j % 4] = q[j % 4] + sv * sv
                plsc.store_scatter(rows_v, [ridx, cj], sv)
            mean = ((s[0] + s[1]) + (s[2] + s[3])) * (1.0 / 64.0)
            var = (((q[0] + q[1]) + (q[2] + q[3])) * (1.0 / 64.0)
                   - mean * mean)
            m1 = _rsqrt16(var + EPS)
            m2 = mean * m1
            for j in range(EMBED):
                cj = jnp.full((16,), j, jnp.int32)
                sv = plsc.load_gather(rows_v, [ridx, cj])
                n = sv * m1 - m2
                o = n * gvec[j // 16][j % 16] + bvec[j // 16][j % 16]
                plsc.store_scatter(rows_v, [ridx, cj], o)
            return carry
        lax.fori_loop(0, NGRP, grp_body, 0)

    def writeback(c, b):
        out_base = wid * ROWS_PER_W + c * CHUNK
        pltpu.sync_copy(rows_v.at[pl.ds(b * CHUNK, CHUNK)],
                        out_hbm.at[pl.ds(out_base, CHUNK)])

    start_gather(0, 0)

    def pair_body(i, carry):
        c2 = i * NBUF
        for b in range(NBUF):
            c = c2 + b

            @pl.when(c < NCHUNK - 1)
            def _():
                start_gather(c + 1, (b + 1) % NBUF)
            drain(b)
            compute(c, b)
            writeback(c, b)
        return carry
    lax.fori_loop(0, NCHUNK // NBUF, pair_body, 0)


@jax.jit
def _sc_call(ids, token_table, posc, gb):
    mesh = plsc.VectorSubcoreMesh(core_axis_name="c", subcore_axis_name="s")
    f = functools.partial(
        pl.kernel,
        mesh=mesh,
        out_type=jax.ShapeDtypeStruct((BL, EMBED), jnp.float32),
        compiler_params=pltpu.CompilerParams(
            needs_layout_passes=False, use_tc_tiling_on_sc=False),
        scratch_types=[
            pltpu.VMEM((NBUF * NG, GSIZE), jnp.int32),
            pltpu.VMEM((NBUF * CHUNK, EMBED), jnp.float32),
            pltpu.VMEM((EMBED, PBUF), jnp.float32),
            pltpu.VMEM((2, EMBED), jnp.float32),
            pltpu.SemaphoreType.DMA,
            pltpu.SemaphoreType.DMA,
        ],
    )(_sc_body)
    return f(ids, token_table, posc, gb)


def kernel(input_ids, token_table, pos_table, ln_gamma, ln_beta):
    ids = input_ids.reshape(BL // GSIZE, GSIZE).astype(jnp.int32)
    pt = pos_table[:L].T
    posc = jnp.concatenate([pt, pt, pt[:, :16]], axis=1)
    gb = jnp.stack([ln_gamma, ln_beta])
    out = _sc_call(ids, token_table, posc, gb)
    return out.reshape(B, L, EMBED)


# speedup vs baseline: 3.5010x; 3.5010x over previous
import functools

import jax
import jax.numpy as jnp
from jax import lax
from jax.experimental import pallas as pl
from jax.experimental.pallas import tpu as pltpu
from jax.experimental.pallas import tpu_sc as plsc

VOCAB = 1000000
EMBED = 64
B = 4096
L = 200
EPS = 1e-5

BL = B * L
NC, NS = 2, 16
NW = NC * NS
ROWS_PER_W = BL // NW
CHUNK = 800
NPER = CHUNK // L
NCHUNK = ROWS_PER_W // CHUNK
GSIZE = 100
NG = CHUNK // GSIZE
NBUF = 2


def _rsqrt16(v):
    i = lax.bitcast_convert_type(v, jnp.int32)
    y = lax.bitcast_convert_type(
        jnp.int32(0x5F3759DF) - lax.shift_right_arithmetic(i, 1), jnp.float32)
    for _ in range(3):
        y = y * (1.5 - 0.5 * v * y * y)
    return y


def _lanesum(v, perms):
    for p in perms:
        v = v + jnp.take(v, p)
    return v


def _sc_body(ids_hbm, tok_hbm, pos_hbm, gb_hbm, out_hbm,
             idx_v, rows_v, pos_v, gb_v, sem0, sem1):
    wid = lax.axis_index("s") * NC + lax.axis_index("c")
    sems = (sem0, sem1)

    pltpu.sync_copy(pos_hbm.at[pl.ds(0, L)], pos_v)
    pltpu.sync_copy(gb_hbm, gb_v)
    g = [gb_v[0, pl.ds(16 * j, 16)] for j in range(4)]
    bt = [gb_v[1, pl.ds(16 * j, 16)] for j in range(4)]
    iota16 = lax.iota(jnp.int32, 16)
    perms = [jnp.bitwise_xor(iota16, jnp.int32(1 << k)) for k in range(4)]

    def start_gather(c, b):
        idx_row = wid * (ROWS_PER_W // GSIZE) + c * NG
        pltpu.sync_copy(ids_hbm.at[pl.ds(idx_row, NG)],
                        idx_v.at[pl.ds(b * NG, NG)])
        for j in range(NG):
            pltpu.async_copy(
                tok_hbm.at[idx_v.at[b * NG + j]],
                rows_v.at[pl.ds(b * CHUNK + j * GSIZE, GSIZE)], sems[b])

    def drain(b):
        pltpu.make_async_copy(
            tok_hbm.at[pl.ds(0, CHUNK)],
            rows_v.at[pl.ds(b * CHUNK, CHUNK)], sems[b]).wait()

    def compute(b):
        base = b * CHUNK

        @plsc.parallel_loop(0, L, unroll=2)
        def col_body(l):
            p0 = [pos_v[l, pl.ds(16 * j, 16)] for j in range(4)]
            for p in range(NPER):
                r = base + p * L + l
                x = [rows_v[r, pl.ds(16 * j, 16)] + p0[j] for j in range(4)]
                s = _lanesum((x[0] + x[1]) + (x[2] + x[3]), perms)
                mean = s * (1.0 / 64.0)
                q = _lanesum((x[0] * x[0] + x[1] * x[1])
                             + (x[2] * x[2] + x[3] * x[3]), perms)
                var = q * (1.0 / 64.0) - mean * mean
                m1 = _rsqrt16(var + EPS)
                m2 = mean * m1
                for j in range(4):
                    rows_v[r, pl.ds(16 * j, 16)] = (
                        (x[j] * m1 - m2) * g[j] + bt[j])

    def writeback(c, b):
        out_base = wid * ROWS_PER_W + c * CHUNK
        pltpu.sync_copy(rows_v.at[pl.ds(b * CHUNK, CHUNK)],
                        out_hbm.at[pl.ds(out_base, CHUNK)])

    start_gather(0, 0)

    def pair_body(i, carry):
        c2 = i * NBUF
        for b in range(NBUF):
            c = c2 + b

            @pl.when(c < NCHUNK - 1)
            def _():
                start_gather(c + 1, (b + 1) % NBUF)
            drain(b)
            compute(b)
            writeback(c, b)
        return carry
    lax.fori_loop(0, NCHUNK // NBUF, pair_body, 0)


@jax.jit
def _sc_call(ids, token_table, pos_table, gb):
    mesh = plsc.VectorSubcoreMesh(core_axis_name="c", subcore_axis_name="s")
    f = functools.partial(
        pl.kernel,
        mesh=mesh,
        out_type=jax.ShapeDtypeStruct((BL, EMBED), jnp.float32),
        compiler_params=pltpu.CompilerParams(
            needs_layout_passes=False, use_tc_tiling_on_sc=False),
        scratch_types=[
            pltpu.VMEM((NBUF * NG, GSIZE), jnp.int32),
            pltpu.VMEM((NBUF * CHUNK, EMBED), jnp.float32),
            pltpu.VMEM((L, EMBED), jnp.float32),
            pltpu.VMEM((2, EMBED), jnp.float32),
            pltpu.SemaphoreType.DMA,
            pltpu.SemaphoreType.DMA,
        ],
    )(_sc_body)
    return f(ids, token_table, pos_table, gb)


def kernel(input_ids, token_table, pos_table, ln_gamma, ln_beta):
    ids = input_ids.reshape(BL // GSIZE, GSIZE).astype(jnp.int32)
    gb = jnp.stack([ln_gamma, ln_beta])
    out = _sc_call(ids, token_table, pos_table, gb)
    return out.reshape(B, L, EMBED)
